# baseline (speedup 1.0000x reference)
import jax
import jax.numpy as jnp
from jax import lax
from jax.experimental import pallas as pl
from jax.experimental.pallas import tpu as pltpu
from jax.experimental.pallas import tpu_sc as plsc

NNZ = 327680
BATCH = 16384
NUM_VALUES = 100001
NUM_FEATURES = 64

NC = 2
NS = 16
NW = NC * NS
L = 16
QF = NUM_FEATURES // L

IB = 128
ROWS_TOTAL = NNZ // IB
NCHUNK = ROWS_TOTAL // NW
NBUF = 4
NIDX = 8
ROWS_PER_TILE = BATCH // NS


def _sc_body(matrix, colh, rowh, cnth, out_e, out_c,
             acc_e, acc_c,
             col_r, row_r, cnt_r,
             g0, g1, g2, g3, c0, c1, c2, c3,
             gs0, gs1, gs2, gs3, ss0, ss1, ss2, ss3,
             is0, is1, is2, is3, is4, is5, is6, is7):
    gb = [g0, g1, g2, g3]
    cs = [c0, c1, c2, c3]
    gsem = [gs0, gs1, gs2, gs3]
    ssem = [ss0, ss1, ss2, ss3]
    isem = [is0, is1, is2, is3, is4, is5, is6, is7]

    cid = lax.axis_index("c")
    sid = lax.axis_index("s")
    w = sid * NC + cid
    base_r = w * NCHUNK

    zero16 = jnp.zeros((L,), jnp.float32)
    ones = jnp.ones((L,), jnp.float32)

    def _zero(i, _):
        for q in range(QF):
            gb[0][i, pl.ds(q * L, L)] = zero16
        return 0
    lax.fori_loop(0, IB, _zero, 0)

    def _zero_c(g, _):
        cs[0][pl.ds(g * L, L)] = zero16
        return 0
    lax.fori_loop(0, IB // L, _zero_c, 0)

    for k in range(ROWS_PER_TILE // IB):
        base = sid * ROWS_PER_TILE + k * IB
        pltpu.sync_copy(gb[0], acc_e.at[pl.ds(base, IB)])
        pltpu.sync_copy(cs[0], acc_c.at[pl.ds(base, IB)])
    plsc.subcore_barrier()

    def fire_idx(t, s):
        pltpu.async_copy(colh.at[base_r + t], col_r.at[s], isem[s])
        pltpu.async_copy(rowh.at[base_r + t], row_r.at[s], isem[s])
        pltpu.async_copy(cnth.at[base_r + t], cnt_r.at[s], isem[s])

    def wait_idx(s):
        pltpu.make_async_copy(colh.at[0], col_r.at[s], isem[s]).wait()
        pltpu.make_async_copy(rowh.at[0], row_r.at[s], isem[s]).wait()
        pltpu.make_async_copy(cnth.at[0], cnt_r.at[s], isem[s]).wait()

    def fire_gather(b, s):
        pltpu.async_copy(matrix.at[col_r.at[s]], gb[b], gsem[b])

    def wait_gather(b):
        pltpu.make_async_copy(matrix.at[col_r.at[0]], gb[b],
                              gsem[b]).wait()

    def fire_scatter(b, s):
        pltpu.async_copy(gb[b], acc_e.at[row_r.at[s]], ssem[b], add=True)
        pltpu.async_copy(cs[b], acc_c.at[row_r.at[s]], ssem[b], add=True)

    def wait_scatter(b):
        pltpu.make_async_copy(gb[b], acc_e.at[row_r.at[0]], ssem[b]).wait()
        pltpu.make_async_copy(cs[b], acc_c.at[row_r.at[0]], ssem[b]).wait()

    def scale(b, s):
        def sb(g, _):
            cv = cnt_r[s, pl.ds(g * L, L)]
            r0 = g * L
            cs[b][pl.ds(r0, L)] = cv
            for e in range(L):
                c = cv[e] * ones
                for q in range(QF):
                    gb[b][r0 + e, pl.ds(q * L, L)] = (
                        gb[b][r0 + e, pl.ds(q * L, L)] * c)
            return 0
        lax.fori_loop(0, IB // L, sb, 0)

    fire_idx(0, 0)
    fire_idx(1, 1)
    wait_idx(0)
    fire_gather(0, 0)

    def body(gidx, _):
        for u in range(NIDX):
            t = gidx * NIDX + u
            b = u % NBUF
            nb = (u + 1) % NBUF
            s = u
            ns = (u + 1) % NIDX
            ps = (u + 2) % NIDX

            @pl.when(t >= NBUF - 1)
            def _():
                wait_scatter(nb)

            @pl.when(t + 2 < NCHUNK)
            def _():
                fire_idx(t + 2, ps)

            @pl.when(t + 1 < NCHUNK)
            def _():
                wait_idx(ns)
                fire_gather(nb, ns)

            wait_gather(b)
            scale(b, s)
            fire_scatter(b, s)
        return 0
    lax.fori_loop(0, NCHUNK // NIDX, body, 0)

    for t in range(NCHUNK - NBUF + 1, NCHUNK):
        wait_scatter(t % NBUF)

    plsc.subcore_barrier()
    base = sid * ROWS_PER_TILE
    pltpu.sync_copy(acc_e.at[pl.ds(base, ROWS_PER_TILE)],
                    out_e.at[cid].at[pl.ds(base, ROWS_PER_TILE)])
    pltpu.sync_copy(acc_c.at[pl.ds(base, ROWS_PER_TILE)],
                    out_c.at[cid].at[pl.ds(base, ROWS_PER_TILE)])


def _combine_body(pe_ref, pc_ref, o_ref):
    e = pe_ref[0] + pe_ref[1]
    t = pc_ref[0] + pc_ref[1]
    o_ref[...] = e / t


@jax.jit
def kernel(matrix, counts, row_ids, col_ids):
    colh = col_ids.astype(jnp.int32).reshape(ROWS_TOTAL, IB)
    rowh = row_ids.astype(jnp.int32).reshape(ROWS_TOTAL, IB)
    cnth = counts.reshape(ROWS_TOTAL, IB)

    mesh = plsc.VectorSubcoreMesh(core_axis_name="c", subcore_axis_name="s")
    sc = pl.kernel(
        _sc_body,
        out_type=[
            jax.ShapeDtypeStruct((NC, BATCH, NUM_FEATURES), jnp.float32),
            jax.ShapeDtypeStruct((NC, BATCH), jnp.float32),
        ],
        mesh=mesh,
        compiler_params=pltpu.CompilerParams(
            use_tc_tiling_on_sc=False, needs_layout_passes=False),
        scratch_types=(
            [
                pltpu.VMEM_SHARED((BATCH, NUM_FEATURES), jnp.float32),
                pltpu.VMEM_SHARED((BATCH,), jnp.float32),
                pltpu.VMEM((NIDX, IB), jnp.int32),
                pltpu.VMEM((NIDX, IB), jnp.int32),
                pltpu.VMEM((NIDX, IB), jnp.float32),
            ]
            + [pltpu.VMEM((IB, NUM_FEATURES), jnp.float32)] * NBUF
            + [pltpu.VMEM((IB,), jnp.float32)] * NBUF
            + [pltpu.SemaphoreType.DMA] * (2 * NBUF + NIDX)
        ),
    )
    part_e, part_c = sc(matrix, colh, rowh, cnth)
    part_c = part_c.reshape(NC, BATCH, 1)

    rows_blk = 1024
    out = pl.pallas_call(
        _combine_body,
        grid=(BATCH // rows_blk,),
        in_specs=[
            pl.BlockSpec((NC, rows_blk, NUM_FEATURES), lambda i: (0, i, 0)),
            pl.BlockSpec((NC, rows_blk, 1), lambda i: (0, i, 0)),
        ],
        out_specs=pl.BlockSpec((rows_blk, NUM_FEATURES), lambda i: (i, 0)),
        out_shape=jax.ShapeDtypeStruct((BATCH, NUM_FEATURES), jnp.float32),
    )(part_e, part_c)
    return out


# speedup vs baseline: 1.0004x; 1.0004x over previous
import jax
import jax.numpy as jnp
from jax import lax
from jax.experimental import pallas as pl
from jax.experimental.pallas import tpu as pltpu
from jax.experimental.pallas import tpu_sc as plsc

NNZ = 327680
BATCH = 16384
NUM_VALUES = 100001
NUM_FEATURES = 64

NC = 2
NS = 16
NW = NC * NS
L = 16
QF = NUM_FEATURES // L

IB = 128
ROWS_TOTAL = NNZ // IB
NCHUNK = ROWS_TOTAL // NW
NBUF = 4
NIDX = 8
ROWS_PER_TILE = BATCH // NS


def _sc_body(matrix, colh, rowh, cnth, out_e,
             acc_e, acc_c,
             col_r, row_r, cnt_r, mrow_r, mcnt_r, tbuf,
             g0, g1, g2, g3, c0, c1, c2, c3, m0, m1, m2, m3,
             gs0, gs1, gs2, gs3, ss0, ss1, ss2, ss3,
             is0, is1, is2, is3, is4, is5, is6, is7,
             mi0, mi1, mi2, mi3, mi4, mi5, mi6, mi7,
             ms0, ms1, ms2, ms3):
    gb = [g0, g1, g2, g3]
    cs = [c0, c1, c2, c3]
    mcs = [m0, m1, m2, m3]
    gsem = [gs0, gs1, gs2, gs3]
    ssem = [ss0, ss1, ss2, ss3]
    isem = [is0, is1, is2, is3, is4, is5, is6, is7]
    misem = [mi0, mi1, mi2, mi3, mi4, mi5, mi6, mi7]
    mssem = [ms0, ms1, ms2, ms3]

    cid = lax.axis_index("c")
    sid = lax.axis_index("s")
    w = sid * NC + cid
    mw = sid * NC + (1 - cid)
    base_r = w * NCHUNK
    mbase_r = mw * NCHUNK

    zero16 = jnp.zeros((L,), jnp.float32)
    ones = jnp.ones((L,), jnp.float32)

    def _zero(i, _):
        for q in range(QF):
            gb[0][i, pl.ds(q * L, L)] = zero16
        return 0
    lax.fori_loop(0, IB, _zero, 0)

    def _zero_c(g, _):
        cs[0][pl.ds(g * L, L)] = zero16
        return 0
    lax.fori_loop(0, IB // L, _zero_c, 0)

    for k in range(ROWS_PER_TILE // IB):
        base = sid * ROWS_PER_TILE + k * IB
        pltpu.sync_copy(gb[0], acc_e.at[pl.ds(base, IB)])
        pltpu.sync_copy(cs[0], acc_c.at[pl.ds(base, IB)])
    plsc.subcore_barrier()

    def fire_idx(t, s):
        pltpu.async_copy(colh.at[base_r + t], col_r.at[s], isem[s])
        pltpu.async_copy(rowh.at[base_r + t], row_r.at[s], isem[s])
        pltpu.async_copy(cnth.at[base_r + t], cnt_r.at[s], isem[s])

    def wait_idx(s):
        pltpu.make_async_copy(colh.at[0], col_r.at[s], isem[s]).wait()
        pltpu.make_async_copy(rowh.at[0], row_r.at[s], isem[s]).wait()
        pltpu.make_async_copy(cnth.at[0], cnt_r.at[s], isem[s]).wait()

    def fire_midx(t, s):
        pltpu.async_copy(rowh.at[mbase_r + t], mrow_r.at[s], misem[s])
        pltpu.async_copy(cnth.at[mbase_r + t], mcnt_r.at[s], misem[s])

    def wait_midx(s):
        pltpu.make_async_copy(rowh.at[0], mrow_r.at[s], misem[s]).wait()
        pltpu.make_async_copy(cnth.at[0], mcnt_r.at[s], misem[s]).wait()

    def fire_mscatter(b, s):
        pltpu.async_copy(mcs[b], acc_c.at[mrow_r.at[s]], mssem[b],
                         add=True)

    def wait_mscatter(b):
        pltpu.make_async_copy(mcs[b], acc_c.at[mrow_r.at[0]],
                              mssem[b]).wait()

    def fire_gather(b, s):
        pltpu.async_copy(matrix.at[col_r.at[s]], gb[b], gsem[b])

    def wait_gather(b):
        pltpu.make_async_copy(matrix.at[col_r.at[0]], gb[b],
                              gsem[b]).wait()

    def fire_scatter(b, s):
        pltpu.async_copy(gb[b], acc_e.at[row_r.at[s]], ssem[b], add=True)
        pltpu.async_copy(cs[b], acc_c.at[row_r.at[s]], ssem[b], add=True)

    def wait_scatter(b):
        pltpu.make_async_copy(gb[b], acc_e.at[row_r.at[0]], ssem[b]).wait()
        pltpu.make_async_copy(cs[b], acc_c.at[row_r.at[0]], ssem[b]).wait()

    def scale(b, s):
        def sb(g, _):
            cv = cnt_r[s, pl.ds(g * L, L)]
            r0 = g * L
            cs[b][pl.ds(r0, L)] = cv
            for e in range(L):
                c = cv[e] * ones
                for q in range(QF):
                    gb[b][r0 + e, pl.ds(q * L, L)] = (
                        gb[b][r0 + e, pl.ds(q * L, L)] * c)
            return 0
        lax.fori_loop(0, IB // L, sb, 0)

    fire_idx(0, 0)
    fire_idx(1, 1)
    fire_midx(0, 0)
    fire_midx(1, 1)
    wait_idx(0)
    fire_gather(0, 0)

    def body(gidx, _):
        for u in range(NIDX):
            t = gidx * NIDX + u
            b = u % NBUF
            nb = (u + 1) % NBUF
            s = u
            ns = (u + 1) % NIDX
            ps = (u + 2) % NIDX

            @pl.when(t >= NBUF - 1)
            def _():
                wait_scatter(nb)

            @pl.when(t >= 2)
            def _():
                wait_mscatter((u + 2) % NBUF)

            @pl.when(t + 2 < NCHUNK)
            def _():
                fire_idx(t + 2, ps)
                fire_midx(t + 2, ps)

            @pl.when(t + 1 < NCHUNK)
            def _():
                wait_idx(ns)
                fire_gather(nb, ns)

            wait_midx(s)

            def mc(g, _):
                mcs[b][pl.ds(g * L, L)] = mcnt_r[s, pl.ds(g * L, L)]
                return 0
            lax.fori_loop(0, IB // L, mc, 0)
            fire_mscatter(b, s)

            wait_gather(b)
            scale(b, s)
            fire_scatter(b, s)
        return 0
    lax.fori_loop(0, NCHUNK // NIDX, body, 0)

    for t in range(NCHUNK - NBUF + 1, NCHUNK):
        wait_scatter(t % NBUF)
    for t in range(NCHUNK - 2, NCHUNK):
        wait_mscatter(t % NBUF)

    plsc.subcore_barrier()

    dbase = sid * ROWS_PER_TILE
    pltpu.sync_copy(acc_c.at[pl.ds(dbase, ROWS_PER_TILE)], tbuf)
    for k in range(ROWS_PER_TILE // IB):
        b = k % NBUF
        pltpu.sync_copy(acc_e.at[pl.ds(dbase + k * IB, IB)], gb[b])

        def dv(g, _):
            tv = tbuf[pl.ds(k * IB + g * L, L)]
            rv = ones / tv
            r0 = g * L
            for e in range(L):
                c = rv[e] * ones
                for q in range(QF):
                    gb[b][r0 + e, pl.ds(q * L, L)] = (
                        gb[b][r0 + e, pl.ds(q * L, L)] * c)
            return 0
        lax.fori_loop(0, IB // L, dv, 0)
        pltpu.sync_copy(gb[b],
                        out_e.at[cid].at[pl.ds(dbase + k * IB, IB)])


def _combine_body(pe_ref, o_ref):
    o_ref[...] = pe_ref[0] + pe_ref[1]


@jax.jit
def kernel(matrix, counts, row_ids, col_ids):
    colh = col_ids.astype(jnp.int32).reshape(ROWS_TOTAL, IB)
    rowh = row_ids.astype(jnp.int32).reshape(ROWS_TOTAL, IB)
    cnth = counts.reshape(ROWS_TOTAL, IB)

    mesh = plsc.VectorSubcoreMesh(core_axis_name="c", subcore_axis_name="s")
    sc = pl.kernel(
        _sc_body,
        out_type=jax.ShapeDtypeStruct((NC, BATCH, NUM_FEATURES),
                                      jnp.float32),
        mesh=mesh,
        compiler_params=pltpu.CompilerParams(
            use_tc_tiling_on_sc=False, needs_layout_passes=False),
        scratch_types=(
            [
                pltpu.VMEM_SHARED((BATCH, NUM_FEATURES), jnp.float32),
                pltpu.VMEM_SHARED((BATCH,), jnp.float32),
                pltpu.VMEM((NIDX, IB), jnp.int32),
                pltpu.VMEM((NIDX, IB), jnp.int32),
                pltpu.VMEM((NIDX, IB), jnp.float32),
                pltpu.VMEM((NIDX, IB), jnp.int32),
                pltpu.VMEM((NIDX, IB), jnp.float32),
                pltpu.VMEM((ROWS_PER_TILE,), jnp.float32),
            ]
            + [pltpu.VMEM((IB, NUM_FEATURES), jnp.float32)] * NBUF
            + [pltpu.VMEM((IB,), jnp.float32)] * NBUF
            + [pltpu.VMEM((IB,), jnp.float32)] * NBUF
            + [pltpu.SemaphoreType.DMA] * (2 * NBUF + 2 * NIDX + NBUF)
        ),
    )
    part_e = sc(matrix, colh, rowh, cnth)

    rows_blk = 1024
    out = pl.pallas_call(
        _combine_body,
        grid=(BATCH // rows_blk,),
        in_specs=[
            pl.BlockSpec((NC, rows_blk, NUM_FEATURES), lambda i: (0, i, 0)),
        ],
        out_specs=pl.BlockSpec((rows_blk, NUM_FEATURES), lambda i: (i, 0)),
        out_shape=jax.ShapeDtypeStruct((BATCH, NUM_FEATURES), jnp.float32),
    )(part_e)
    return out


# speedup vs baseline: 1.2869x; 1.2864x over previous
import jax
import jax.numpy as jnp
from jax import lax
from jax.experimental import pallas as pl
from jax.experimental.pallas import tpu as pltpu
from jax.experimental.pallas import tpu_sc as plsc

NNZ = 327680
BATCH = 16384
NUM_VALUES = 100001
NUM_FEATURES = 64

NC = 2
NS = 16
NW = NC * NS
L = 16
QF = NUM_FEATURES // L

IB = 128
ROWS_TOTAL = NNZ // IB
NCHUNK = ROWS_TOTAL // NW
NBUF = 2
NIDX = 4
ROWS_PER_TILE = BATCH // NS
NPIECE = ROWS_PER_TILE // IB


def _sc_body(matrix, colh, rowh, cnth, out_e,
             acc_e, acc_c,
             col_r, row_r, cnt_r, mrow_r, mcnt_r, tbuf,
             g0, g1, b0, b1, c0, c1, m0, m1,
             gs0, gs1, ss0, ss1, ms0, ms1,
             is0, is1, is2, is3,
             mi0, mi1, mi2, mi3):
    gb = [g0, g1]
    sb = [b0, b1]
    cs = [c0, c1]
    mcs = [m0, m1]
    gsem = [gs0, gs1]
    ssem = [ss0, ss1]
    mssem = [ms0, ms1]
    isem = [is0, is1, is2, is3]
    misem = [mi0, mi1, mi2, mi3]

    cid = lax.axis_index("c")
    sid = lax.axis_index("s")
    w = sid * NC + cid
    mw = sid * NC + (1 - cid)
    base_r = w * NCHUNK
    mbase_r = mw * NCHUNK

    zero16 = jnp.zeros((L,), jnp.float32)
    ones = jnp.ones((L,), jnp.float32)

    @plsc.parallel_loop(0, IB)
    def _(i):
        for q in range(QF):
            gb[0][i, pl.ds(q * L, L)] = zero16

    @plsc.parallel_loop(0, IB // L)
    def _(g):
        cs[0][pl.ds(g * L, L)] = zero16

    for k in range(NPIECE):
        base = sid * ROWS_PER_TILE + k * IB
        pltpu.sync_copy(gb[0], acc_e.at[pl.ds(base, IB)])
        pltpu.sync_copy(cs[0], acc_c.at[pl.ds(base, IB)])
    plsc.subcore_barrier()

    def fire_idx(t, s):
        pltpu.async_copy(colh.at[base_r + t], col_r.at[s], isem[s])
        pltpu.async_copy(rowh.at[base_r + t], row_r.at[s], isem[s])
        pltpu.async_copy(cnth.at[base_r + t], cnt_r.at[s], isem[s])

    def wait_idx(s):
        pltpu.make_async_copy(colh.at[0], col_r.at[s], isem[s]).wait()
        pltpu.make_async_copy(rowh.at[0], row_r.at[s], isem[s]).wait()
        pltpu.make_async_copy(cnth.at[0], cnt_r.at[s], isem[s]).wait()

    def fire_midx(t, s):
        pltpu.async_copy(rowh.at[mbase_r + t], mrow_r.at[s], misem[s])
        pltpu.async_copy(cnth.at[mbase_r + t], mcnt_r.at[s], misem[s])

    def wait_midx(s):
        pltpu.make_async_copy(rowh.at[0], mrow_r.at[s], misem[s]).wait()
        pltpu.make_async_copy(cnth.at[0], mcnt_r.at[s], misem[s]).wait()

    def fire_mscatter(b, s):
        pltpu.async_copy(mcs[b], acc_c.at[mrow_r.at[s]], mssem[b],
                         add=True)

    def wait_mscatter(b):
        pltpu.make_async_copy(mcs[b], acc_c.at[mrow_r.at[0]],
                              mssem[b]).wait()

    def fire_gather(b, s):
        pltpu.async_copy(matrix.at[col_r.at[s]], gb[b], gsem[b])

    def wait_gather(b):
        pltpu.make_async_copy(matrix.at[col_r.at[0]], gb[b],
                              gsem[b]).wait()

    def fire_scatter(b, s):
        pltpu.async_copy(sb[b], acc_e.at[row_r.at[s]], ssem[b], add=True)
        pltpu.async_copy(cs[b], acc_c.at[row_r.at[s]], ssem[b], add=True)

    def wait_scatter(b):
        pltpu.make_async_copy(sb[b], acc_e.at[row_r.at[0]], ssem[b]).wait()
        pltpu.make_async_copy(cs[b], acc_c.at[row_r.at[0]], ssem[b]).wait()

    def scale(b, s):
        @plsc.parallel_loop(0, IB // L)
        def _(g):
            cv = cnt_r[s, pl.ds(g * L, L)]
            cs[b][pl.ds(g * L, L)] = cv
            for e in range(L):
                c = cv[e] * ones
                r = g * L + e
                for q in range(QF):
                    sb[b][r, pl.ds(q * L, L)] = (
                        gb[b][r, pl.ds(q * L, L)] * c)

    fire_idx(0, 0)
    fire_idx(1, 1)
    fire_midx(0, 0)
    fire_midx(1, 1)
    wait_idx(0)
    fire_gather(0, 0)

    def body(gidx, _):
        for u in range(NIDX):
            t = gidx * NIDX + u
            b = u % NBUF
            nb = (u + 1) % NBUF
            s = u
            ns = (u + 1) % NIDX
            ps = (u + 2) % NIDX

            @pl.when(t >= NBUF)
            def _():
                wait_scatter(b)
                wait_mscatter(b)

            @pl.when(t + 2 < NCHUNK)
            def _():
                fire_idx(t + 2, ps)
                fire_midx(t + 2, ps)

            @pl.when(t + 1 < NCHUNK)
            def _():
                wait_idx(ns)
                fire_gather(nb, ns)

            wait_midx(s)

            @plsc.parallel_loop(0, IB // L)
            def _(g):
                mcs[b][pl.ds(g * L, L)] = mcnt_r[s, pl.ds(g * L, L)]
            fire_mscatter(b, s)

            wait_gather(b)
            scale(b, s)
            fire_scatter(b, s)
        return 0
    lax.fori_loop(0, NCHUNK // NIDX, body, 0)

    for t in range(NCHUNK - NBUF, NCHUNK):
        wait_scatter(t % NBUF)
        wait_mscatter(t % NBUF)

    plsc.subcore_barrier()

    dbase = sid * ROWS_PER_TILE
    pltpu.sync_copy(acc_c.at[pl.ds(dbase, ROWS_PER_TILE)], tbuf)

    def dump_pair(j, _):
        for b in range(NBUF):
            k = j * NBUF + b
            pltpu.sync_copy(acc_e.at[pl.ds(dbase + k * IB, IB)], gb[b])

            @plsc.parallel_loop(0, IB // L)
            def _(g):
                tv = tbuf[pl.ds(k * IB + g * L, L)]
                rv = ones / tv
                for e in range(L):
                    c = rv[e] * ones
                    r = g * L + e
                    for q in range(QF):
                        sb[b][r, pl.ds(q * L, L)] = (
                            gb[b][r, pl.ds(q * L, L)] * c)
            pltpu.sync_copy(sb[b],
                            out_e.at[cid].at[pl.ds(dbase + k * IB, IB)])
        return 0
    lax.fori_loop(0, NPIECE // NBUF, dump_pair, 0)


def _combine_body(pe_ref, o_ref):
    o_ref[...] = pe_ref[0] + pe_ref[1]


@jax.jit
def kernel(matrix, counts, row_ids, col_ids):
    colh = col_ids.astype(jnp.int32).reshape(ROWS_TOTAL, IB)
    rowh = row_ids.astype(jnp.int32).reshape(ROWS_TOTAL, IB)
    cnth = counts.reshape(ROWS_TOTAL, IB)

    mesh = plsc.VectorSubcoreMesh(core_axis_name="c", subcore_axis_name="s")
    sc = pl.kernel(
        _sc_body,
        out_type=jax.ShapeDtypeStruct((NC, BATCH, NUM_FEATURES),
                                      jnp.float32),
        mesh=mesh,
        compiler_params=pltpu.CompilerParams(
            use_tc_tiling_on_sc=False, needs_layout_passes=False),
        scratch_types=(
            [
                pltpu.VMEM_SHARED((BATCH, NUM_FEATURES), jnp.float32),
                pltpu.VMEM_SHARED((BATCH,), jnp.float32),
                pltpu.VMEM((NIDX, IB), jnp.int32),
                pltpu.VMEM((NIDX, IB), jnp.int32),
                pltpu.VMEM((NIDX, IB), jnp.float32),
                pltpu.VMEM((NIDX, IB), jnp.int32),
                pltpu.VMEM((NIDX, IB), jnp.float32),
                pltpu.VMEM((ROWS_PER_TILE,), jnp.float32),
            ]
            + [pltpu.VMEM((IB, NUM_FEATURES), jnp.float32)] * NBUF
            + [pltpu.VMEM((IB, NUM_FEATURES), jnp.float32)] * NBUF
            + [pltpu.VMEM((IB,), jnp.float32)] * NBUF
            + [pltpu.VMEM((IB,), jnp.float32)] * NBUF
            + [pltpu.SemaphoreType.DMA] * (3 * NBUF + 2 * NIDX)
        ),
    )
    part_e = sc(matrix, colh, rowh, cnth)

    rows_blk = 1024
    out = pl.pallas_call(
        _combine_body,
        grid=(BATCH // rows_blk,),
        in_specs=[
            pl.BlockSpec((NC, rows_blk, NUM_FEATURES), lambda i: (0, i, 0)),
        ],
        out_specs=pl.BlockSpec((rows_blk, NUM_FEATURES), lambda i: (i, 0)),
        out_shape=jax.ShapeDtypeStruct((BATCH, NUM_FEATURES), jnp.float32),
    )(part_e)
    return out


# speedup vs baseline: 1.3483x; 1.0478x over previous
import jax
import jax.numpy as jnp
from jax import lax
from jax.experimental import pallas as pl
from jax.experimental.pallas import tpu as pltpu
from jax.experimental.pallas import tpu_sc as plsc

NNZ = 327680
BATCH = 16384
NUM_VALUES = 100001
NUM_FEATURES = 64

NC = 2
NS = 16
NW = NC * NS
L = 16
QF = NUM_FEATURES // L

IB = 128
ROWS_TOTAL = NNZ // IB
NCHUNK = ROWS_TOTAL // NW
NBUF = 2
NIDX = 4
ROWS_PER_TILE = BATCH // NS
NPIECE = ROWS_PER_TILE // IB


def _sc_body(matrix, colh, rowh, cnth, out_e,
             acc_e, acc_c,
             col_r, row_r, cnt_r, mrow_r, mcnt_r, tbuf, db,
             g0, g1, b0, b1, c0, c1, m0, m1,
             gs0, gs1, ss0, ss1, ms0, ms1,
             is0, is1, is2, is3,
             mi0, mi1, mi2, mi3):
    gb = [g0, g1]
    sb = [b0, b1]
    cs = [c0, c1]
    mcs = [m0, m1]
    gsem = [gs0, gs1]
    ssem = [ss0, ss1]
    mssem = [ms0, ms1]
    isem = [is0, is1, is2, is3]
    misem = [mi0, mi1, mi2, mi3]

    cid = lax.axis_index("c")
    sid = lax.axis_index("s")
    w = sid * NC + cid
    mw = sid * NC + (1 - cid)
    base_r = w * NCHUNK
    mbase_r = mw * NCHUNK

    zero16 = jnp.zeros((L,), jnp.float32)
    ones = jnp.ones((L,), jnp.float32)

    @plsc.parallel_loop(0, IB)
    def _(i):
        for q in range(QF):
            gb[0][i, pl.ds(q * L, L)] = zero16

    @plsc.parallel_loop(0, IB // L)
    def _(g):
        cs[0][pl.ds(g * L, L)] = zero16

    for k in range(NPIECE):
        base = sid * ROWS_PER_TILE + k * IB
        pltpu.sync_copy(gb[0], acc_e.at[pl.ds(base, IB)])
        pltpu.sync_copy(cs[0], acc_c.at[pl.ds(base, IB)])
    plsc.subcore_barrier()

    def fire_idx(t, s):
        pltpu.async_copy(colh.at[base_r + t], col_r.at[s], isem[s])
        pltpu.async_copy(rowh.at[base_r + t], row_r.at[s], isem[s])
        pltpu.async_copy(cnth.at[base_r + t], cnt_r.at[s], isem[s])

    def wait_idx(s):
        pltpu.make_async_copy(colh.at[0], col_r.at[s], isem[s]).wait()
        pltpu.make_async_copy(rowh.at[0], row_r.at[s], isem[s]).wait()
        pltpu.make_async_copy(cnth.at[0], cnt_r.at[s], isem[s]).wait()

    def fire_midx(t, s):
        pltpu.async_copy(rowh.at[mbase_r + t], mrow_r.at[s], misem[s])
        pltpu.async_copy(cnth.at[mbase_r + t], mcnt_r.at[s], misem[s])

    def wait_midx(s):
        pltpu.make_async_copy(rowh.at[0], mrow_r.at[s], misem[s]).wait()
        pltpu.make_async_copy(cnth.at[0], mcnt_r.at[s], misem[s]).wait()

    def fire_mscatter(b, s):
        pltpu.async_copy(mcs[b], acc_c.at[mrow_r.at[s]], mssem[b],
                         add=True)

    def wait_mscatter(b):
        pltpu.make_async_copy(mcs[b], acc_c.at[mrow_r.at[0]],
                              mssem[b]).wait()

    def fire_gather(b, s):
        pltpu.async_copy(matrix.at[col_r.at[s]], gb[b], gsem[b])

    def wait_gather(b):
        pltpu.make_async_copy(matrix.at[col_r.at[0]], gb[b],
                              gsem[b]).wait()

    def fire_scatter(b, s):
        pltpu.async_copy(sb[b], acc_e.at[row_r.at[s]], ssem[b], add=True)
        pltpu.async_copy(cs[b], acc_c.at[row_r.at[s]], ssem[b], add=True)

    def wait_scatter(b):
        pltpu.make_async_copy(sb[b], acc_e.at[row_r.at[0]], ssem[b]).wait()
        pltpu.make_async_copy(cs[b], acc_c.at[row_r.at[0]], ssem[b]).wait()

    def scale(b, s):
        @plsc.parallel_loop(0, IB // L)
        def _(g):
            cv = cnt_r[s, pl.ds(g * L, L)]
            cs[b][pl.ds(g * L, L)] = cv
            for e in range(L):
                c = cv[e] * ones
                r = g * L + e
                for q in range(QF):
                    sb[b][r, pl.ds(q * L, L)] = (
                        gb[b][r, pl.ds(q * L, L)] * c)

    fire_idx(0, 0)
    fire_idx(1, 1)
    fire_midx(0, 0)
    fire_midx(1, 1)
    wait_idx(0)
    fire_gather(0, 0)

    def body(gidx, _):
        for u in range(NIDX):
            t = gidx * NIDX + u
            b = u % NBUF
            nb = (u + 1) % NBUF
            s = u
            ns = (u + 1) % NIDX
            ps = (u + 2) % NIDX

            @pl.when(t >= NBUF)
            def _():
                wait_scatter(b)
                wait_mscatter(b)

            @pl.when(t + 2 < NCHUNK)
            def _():
                fire_idx(t + 2, ps)
                fire_midx(t + 2, ps)

            @pl.when(t + 1 < NCHUNK)
            def _():
                wait_idx(ns)
                fire_gather(nb, ns)

            wait_midx(s)

            @plsc.parallel_loop(0, IB // L)
            def _(g):
                mcs[b][pl.ds(g * L, L)] = mcnt_r[s, pl.ds(g * L, L)]
            fire_mscatter(b, s)

            wait_gather(b)
            scale(b, s)
            fire_scatter(b, s)
        return 0
    lax.fori_loop(0, NCHUNK // NIDX, body, 0)

    for t in range(NCHUNK - NBUF, NCHUNK):
        wait_scatter(t % NBUF)
        wait_mscatter(t % NBUF)

    plsc.subcore_barrier()

    dbase = sid * ROWS_PER_TILE
    pltpu.sync_copy(acc_c.at[pl.ds(dbase, ROWS_PER_TILE)], tbuf)

    def dump_pair(j, _):
        for b in range(NBUF):
            k = j * NBUF + b
            pltpu.sync_copy(acc_e.at[pl.ds(dbase + k * IB, IB)], gb[b])

            @plsc.parallel_loop(0, IB // L)
            def _(g):
                tv = tbuf[pl.ds(k * IB + g * L, L)]
                rv = ones / tv
                for e in range(L):
                    c = rv[e] * ones
                    r = g * L + e
                    for q in range(QF):
                        db[g * 2 + e // 8, e % 8, pl.ds(q * L, L)] = (
                            gb[b][r, pl.ds(q * L, L)] * c)
            pltpu.sync_copy(
                db, out_e.at[cid].at[pl.ds(sid * (ROWS_PER_TILE // 8)
                                           + k * (IB // 8), IB // 8)])
        return 0
    lax.fori_loop(0, NPIECE // NBUF, dump_pair, 0)


def _combine_body(pe_ref, o_ref):
    s = pe_ref[0] + pe_ref[1]
    o_ref[...] = s.reshape(s.shape[0] * 8, 128)[:, :NUM_FEATURES]


@jax.jit
def kernel(matrix, counts, row_ids, col_ids):
    colh = col_ids.astype(jnp.int32).reshape(ROWS_TOTAL, IB)
    rowh = row_ids.astype(jnp.int32).reshape(ROWS_TOTAL, IB)
    cnth = counts.reshape(ROWS_TOTAL, IB)

    mesh = plsc.VectorSubcoreMesh(core_axis_name="c", subcore_axis_name="s")
    sc = pl.kernel(
        _sc_body,
        out_type=jax.ShapeDtypeStruct((NC, BATCH // 8, 8, 128),
                                      jnp.float32),
        mesh=mesh,
        compiler_params=pltpu.CompilerParams(
            use_tc_tiling_on_sc=False, needs_layout_passes=False),
        scratch_types=(
            [
                pltpu.VMEM_SHARED((BATCH, NUM_FEATURES), jnp.float32),
                pltpu.VMEM_SHARED((BATCH,), jnp.float32),
                pltpu.VMEM((NIDX, IB), jnp.int32),
                pltpu.VMEM((NIDX, IB), jnp.int32),
                pltpu.VMEM((NIDX, IB), jnp.float32),
                pltpu.VMEM((NIDX, IB), jnp.int32),
                pltpu.VMEM((NIDX, IB), jnp.float32),
                pltpu.VMEM((ROWS_PER_TILE,), jnp.float32),
                pltpu.VMEM((IB // 8, 8, 128), jnp.float32),
            ]
            + [pltpu.VMEM((IB, NUM_FEATURES), jnp.float32)] * NBUF
            + [pltpu.VMEM((IB, NUM_FEATURES), jnp.float32)] * NBUF
            + [pltpu.VMEM((IB,), jnp.float32)] * NBUF
            + [pltpu.VMEM((IB,), jnp.float32)] * NBUF
            + [pltpu.SemaphoreType.DMA] * (3 * NBUF + 2 * NIDX)
        ),
    )
    part_e = sc(matrix, colh, rowh, cnth)

    rows_blk = 2048
    out = pl.pallas_call(
        _combine_body,
        grid=(BATCH // rows_blk,),
        in_specs=[
            pl.BlockSpec((NC, rows_blk // 8, 8, 128),
                         lambda i: (0, i, 0, 0)),
        ],
        out_specs=pl.BlockSpec((rows_blk, NUM_FEATURES), lambda i: (i, 0)),
        out_shape=jax.ShapeDtypeStruct((BATCH, NUM_FEATURES), jnp.float32),
    )(part_e)
    return out


# speedup vs baseline: 1.5665x; 1.1618x over previous
import jax
import jax.numpy as jnp
from jax import lax
from jax.experimental import pallas as pl
from jax.experimental.pallas import tpu as pltpu
from jax.experimental.pallas import tpu_sc as plsc

NNZ = 327680
BATCH = 16384
NUM_VALUES = 100001
NUM_FEATURES = 64

NC = 2
NS = 16
NW = NC * NS
L = 16
QF = NUM_FEATURES // L

IB = 128
ROWS_TOTAL = NNZ // IB
NCHUNK = ROWS_TOTAL // NW
NBUF = 2
NIDX = 4
ROWS_PER_TILE = BATCH // NS
NPIECE = ROWS_PER_TILE // IB


def _sc_body(matrix, colh, rowh, cnth, out_e,
             acc_e, acc_c,
             col_r, row_r, cnt_r, mrow_r, mcnt_r, tbuf, db,
             g0, g1, b0, b1, c0, c1, m0, m1,
             gs0, gs1, ss0, ss1, ms0, ms1,
             is0, is1, is2, is3,
             mi0, mi1, mi2, mi3):
    gb = [g0, g1]
    sb = [b0, b1]
    cs = [c0, c1]
    mcs = [m0, m1]
    gsem = [gs0, gs1]
    ssem = [ss0, ss1]
    mssem = [ms0, ms1]
    isem = [is0, is1, is2, is3]
    misem = [mi0, mi1, mi2, mi3]

    cid = lax.axis_index("c")
    sid = lax.axis_index("s")
    w = sid * NC + cid
    mw = sid * NC + (1 - cid)
    base_r = w * NCHUNK
    mbase_r = mw * NCHUNK

    zero16 = jnp.zeros((L,), jnp.float32)
    ones = jnp.ones((L,), jnp.float32)

    @plsc.parallel_loop(0, IB)
    def _(i):
        for q in range(QF):
            gb[0][i, pl.ds(q * L, L)] = zero16

    @plsc.parallel_loop(0, IB // L)
    def _(g):
        cs[0][pl.ds(g * L, L)] = zero16

    for k in range(NPIECE):
        base = sid * ROWS_PER_TILE + k * IB
        pltpu.sync_copy(gb[0], acc_e.at[pl.ds(base, IB)])
        pltpu.sync_copy(cs[0], acc_c.at[pl.ds(base, IB)])
    plsc.subcore_barrier()

    def fire_idx(t, s):
        pltpu.async_copy(colh.at[base_r + t], col_r.at[s], isem[s])
        pltpu.async_copy(rowh.at[base_r + t], row_r.at[s], isem[s])
        pltpu.async_copy(cnth.at[base_r + t], cnt_r.at[s], isem[s])

    def wait_idx(s):
        pltpu.make_async_copy(colh.at[0], col_r.at[s], isem[s]).wait()
        pltpu.make_async_copy(rowh.at[0], row_r.at[s], isem[s]).wait()
        pltpu.make_async_copy(cnth.at[0], cnt_r.at[s], isem[s]).wait()

    def fire_midx(t, s):
        pltpu.async_copy(rowh.at[mbase_r + t], mrow_r.at[s], misem[s])
        pltpu.async_copy(cnth.at[mbase_r + t], mcnt_r.at[s], misem[s])

    def wait_midx(s):
        pltpu.make_async_copy(rowh.at[0], mrow_r.at[s], misem[s]).wait()
        pltpu.make_async_copy(cnth.at[0], mcnt_r.at[s], misem[s]).wait()

    def fire_mscatter(b, s):
        pltpu.async_copy(mcs[b], acc_c.at[mrow_r.at[s]], mssem[b],
                         add=True)

    def wait_mscatter(b):
        pltpu.make_async_copy(mcs[b], acc_c.at[mrow_r.at[0]],
                              mssem[b]).wait()

    def fire_gather(b, s):
        pltpu.async_copy(matrix.at[col_r.at[s]], gb[b], gsem[b])

    def wait_gather(b):
        pltpu.make_async_copy(matrix.at[col_r.at[0]], gb[b],
                              gsem[b]).wait()

    def fire_scatter(b, s):
        pltpu.async_copy(sb[b], acc_e.at[row_r.at[s]], ssem[b], add=True)
        pltpu.async_copy(cs[b], acc_c.at[row_r.at[s]], ssem[b], add=True)

    def wait_scatter(b):
        pltpu.make_async_copy(sb[b], acc_e.at[row_r.at[0]], ssem[b]).wait()
        pltpu.make_async_copy(cs[b], acc_c.at[row_r.at[0]], ssem[b]).wait()

    def scale(b, s):
        @plsc.parallel_loop(0, IB // L, unroll=2)
        def _(g):
            cv = cnt_r[s, pl.ds(g * L, L)]
            cs[b][pl.ds(g * L, L)] = cv
            for e in range(L):
                c = cv[e] * ones
                r = g * L + e
                for q in range(QF):
                    sb[b][r, pl.ds(q * L, L)] = (
                        gb[b][r, pl.ds(q * L, L)] * c)

    fire_idx(0, 0)
    fire_idx(1, 1)
    fire_midx(0, 0)
    fire_midx(1, 1)
    wait_idx(0)
    fire_gather(0, 0)

    def body(gidx, _):
        for u in range(NIDX):
            t = gidx * NIDX + u
            b = u % NBUF
            nb = (u + 1) % NBUF
            s = u
            ns = (u + 1) % NIDX
            ps = (u + 2) % NIDX

            @pl.when(t >= NBUF)
            def _():
                wait_scatter(b)
                wait_mscatter(b)

            @pl.when(t + 2 < NCHUNK)
            def _():
                fire_idx(t + 2, ps)
                fire_midx(t + 2, ps)

            @pl.when(t + 1 < NCHUNK)
            def _():
                wait_idx(ns)
                fire_gather(nb, ns)

            wait_midx(s)

            @plsc.parallel_loop(0, IB // L)
            def _(g):
                mcs[b][pl.ds(g * L, L)] = mcnt_r[s, pl.ds(g * L, L)]
            fire_mscatter(b, s)

            wait_gather(b)
            scale(b, s)
            fire_scatter(b, s)
        return 0
    lax.fori_loop(0, NCHUNK // NIDX, body, 0)

    for t in range(NCHUNK - NBUF, NCHUNK):
        wait_scatter(t % NBUF)
        wait_mscatter(t % NBUF)

    plsc.subcore_barrier()

    dbase = sid * ROWS_PER_TILE
    pltpu.sync_copy(acc_c.at[pl.ds(dbase, ROWS_PER_TILE)], tbuf)

    def dump_pair(j, _):
        for b in range(NBUF):
            k = j * NBUF + b
            pltpu.sync_copy(acc_e.at[pl.ds(dbase + k * IB, IB)], gb[b])

            @plsc.parallel_loop(0, IB // L)
            def _(g):
                tv = tbuf[pl.ds(k * IB + g * L, L)]
                rv = ones / tv
                for e in range(L):
                    c = rv[e] * ones
                    r = g * L + e
                    for q in range(QF):
                        db[g * 2 + e // 8, e % 8, pl.ds(q * L, L)] = (
                            gb[b][r, pl.ds(q * L, L)] * c)
            pltpu.sync_copy(
                db, out_e.at[cid].at[pl.ds(sid * (ROWS_PER_TILE // 8)
                                           + k * (IB // 8), IB // 8)])
        return 0
    lax.fori_loop(0, NPIECE // NBUF, dump_pair, 0)


def _combine_body(pe_ref, o_ref):
    s = pe_ref[0] + pe_ref[1]
    o_ref[...] = s.reshape(s.shape[0] * 8, 128)[:, :NUM_FEATURES]


@jax.jit
def kernel(matrix, counts, row_ids, col_ids):
    colh = col_ids.astype(jnp.int32).reshape(ROWS_TOTAL, IB)
    rowh = row_ids.astype(jnp.int32).reshape(ROWS_TOTAL, IB)
    cnth = counts.reshape(ROWS_TOTAL, IB)

    mesh = plsc.VectorSubcoreMesh(core_axis_name="c", subcore_axis_name="s")
    sc = pl.kernel(
        _sc_body,
        out_type=jax.ShapeDtypeStruct((NC, BATCH // 8, 8, 128),
                                      jnp.float32),
        mesh=mesh,
        compiler_params=pltpu.CompilerParams(
            use_tc_tiling_on_sc=False, needs_layout_passes=False),
        scratch_types=(
            [
                pltpu.VMEM_SHARED((BATCH, NUM_FEATURES), jnp.float32),
                pltpu.VMEM_SHARED((BATCH,), jnp.float32),
                pltpu.VMEM((NIDX, IB), jnp.int32),
                pltpu.VMEM((NIDX, IB), jnp.int32),
                pltpu.VMEM((NIDX, IB), jnp.float32),
                pltpu.VMEM((NIDX, IB), jnp.int32),
                pltpu.VMEM((NIDX, IB), jnp.float32),
                pltpu.VMEM((ROWS_PER_TILE,), jnp.float32),
                pltpu.VMEM((IB // 8, 8, 128), jnp.float32),
            ]
            + [pltpu.VMEM((IB, NUM_FEATURES), jnp.float32)] * NBUF
            + [pltpu.VMEM((IB, NUM_FEATURES), jnp.float32)] * NBUF
            + [pltpu.VMEM((IB,), jnp.float32)] * NBUF
            + [pltpu.VMEM((IB,), jnp.float32)] * NBUF
            + [pltpu.SemaphoreType.DMA] * (3 * NBUF + 2 * NIDX)
        ),
    )
    part_e = sc(matrix, colh, rowh, cnth)

    rows_blk = 2048
    out = pl.pallas_call(
        _combine_body,
        grid=(BATCH // rows_blk,),
        in_specs=[
            pl.BlockSpec((NC, rows_blk // 8, 8, 128),
                         lambda i: (0, i, 0, 0)),
        ],
        out_specs=pl.BlockSpec((rows_blk, NUM_FEATURES), lambda i: (i, 0)),
        out_shape=jax.ShapeDtypeStruct((BATCH, NUM_FEATURES), jnp.float32),
    )(part_e)
    return out


# speedup vs baseline: 1.6130x; 1.0297x over previous
import jax
import jax.numpy as jnp
from jax import lax
from jax.experimental import pallas as pl
from jax.experimental.pallas import tpu as pltpu
from jax.experimental.pallas import tpu_sc as plsc

NNZ = 327680
BATCH = 16384
NUM_VALUES = 100001
NUM_FEATURES = 64

NC = 2
NS = 16
NW = NC * NS
L = 16
QF = NUM_FEATURES // L

IB = 128
ROWS_TOTAL = NNZ // IB
NCHUNK = ROWS_TOTAL // NW
NBUF = 2
NIDX = 4
ROWS_PER_TILE = BATCH // NS
NPIECE = ROWS_PER_TILE // IB


def _sc_body(matrix, colh, rowh, cnth, out_e,
             acc_e, acc_c,
             col_r, row_r, cnt_r, mrow_r, mcnt_r, tbuf, db,
             g0, g1, b0, b1, c0, c1, m0, m1,
             gs0, gs1, ss0, ss1, ms0, ms1,
             is0, is1, is2, is3,
             mi0, mi1, mi2, mi3):
    gb = [g0, g1]
    sb = [b0, b1]
    cs = [c0, c1]
    mcs = [m0, m1]
    gsem = [gs0, gs1]
    ssem = [ss0, ss1]
    mssem = [ms0, ms1]
    isem = [is0, is1, is2, is3]
    misem = [mi0, mi1, mi2, mi3]

    cid = lax.axis_index("c")
    sid = lax.axis_index("s")
    w = sid * NC + cid
    mw = sid * NC + (1 - cid)
    base_r = w * NCHUNK
    mbase_r = mw * NCHUNK

    zero16 = jnp.zeros((L,), jnp.float32)
    ones = jnp.ones((L,), jnp.float32)

    @plsc.parallel_loop(0, IB)
    def _(i):
        for q in range(QF):
            gb[0][i, pl.ds(q * L, L)] = zero16

    @plsc.parallel_loop(0, IB // L)
    def _(g):
        cs[0][pl.ds(g * L, L)] = zero16

    for k in range(NPIECE):
        base = sid * ROWS_PER_TILE + k * IB
        pltpu.sync_copy(gb[0], acc_e.at[pl.ds(base, IB)])
        pltpu.sync_copy(cs[0], acc_c.at[pl.ds(base, IB)])
    plsc.subcore_barrier()

    def fire_idx(t, s):
        pltpu.async_copy(colh.at[base_r + t], col_r.at[s], isem[s])
        pltpu.async_copy(rowh.at[base_r + t], row_r.at[s], isem[s])
        pltpu.async_copy(cnth.at[base_r + t], cnt_r.at[s], isem[s])

    def wait_idx(s):
        pltpu.make_async_copy(colh.at[0], col_r.at[s], isem[s]).wait()
        pltpu.make_async_copy(rowh.at[0], row_r.at[s], isem[s]).wait()
        pltpu.make_async_copy(cnth.at[0], cnt_r.at[s], isem[s]).wait()

    def fire_midx(t, s):
        pltpu.async_copy(rowh.at[mbase_r + t], mrow_r.at[s], misem[s])
        pltpu.async_copy(cnth.at[mbase_r + t], mcnt_r.at[s], misem[s])

    def wait_midx(s):
        pltpu.make_async_copy(rowh.at[0], mrow_r.at[s], misem[s]).wait()
        pltpu.make_async_copy(cnth.at[0], mcnt_r.at[s], misem[s]).wait()

    def fire_mscatter(b, s):
        pltpu.async_copy(mcs[b], acc_c.at[mrow_r.at[s]], mssem[b],
                         add=True)

    def wait_mscatter(b):
        pltpu.make_async_copy(mcs[b], acc_c.at[mrow_r.at[0]],
                              mssem[b]).wait()

    def fire_gather(b, s):
        pltpu.async_copy(matrix.at[col_r.at[s]], gb[b], gsem[b])

    def wait_gather(b):
        pltpu.make_async_copy(matrix.at[col_r.at[0]], gb[b],
                              gsem[b]).wait()

    def fire_scatter(b, s):
        pltpu.async_copy(sb[b], acc_e.at[row_r.at[s]], ssem[b], add=True)
        pltpu.async_copy(cs[b], acc_c.at[row_r.at[s]], ssem[b], add=True)

    def wait_scatter(b):
        pltpu.make_async_copy(sb[b], acc_e.at[row_r.at[0]], ssem[b]).wait()
        pltpu.make_async_copy(cs[b], acc_c.at[row_r.at[0]], ssem[b]).wait()

    def scale(b, s):
        @plsc.parallel_loop(0, IB // L, unroll=4)
        def _(g):
            cv = cnt_r[s, pl.ds(g * L, L)]
            cs[b][pl.ds(g * L, L)] = cv
            for e in range(L):
                c = cv[e] * ones
                r = g * L + e
                for q in range(QF):
                    sb[b][r, pl.ds(q * L, L)] = (
                        gb[b][r, pl.ds(q * L, L)] * c)

    fire_idx(0, 0)
    fire_idx(1, 1)
    fire_midx(0, 0)
    fire_midx(1, 1)
    wait_idx(0)
    fire_gather(0, 0)

    def body(gidx, _):
        for u in range(NIDX):
            t = gidx * NIDX + u
            b = u % NBUF
            nb = (u + 1) % NBUF
            s = u
            ns = (u + 1) % NIDX
            ps = (u + 2) % NIDX

            @pl.when(t >= NBUF)
            def _():
                wait_scatter(b)
                wait_mscatter(b)

            @pl.when(t + 2 < NCHUNK)
            def _():
                fire_idx(t + 2, ps)
                fire_midx(t + 2, ps)

            @pl.when(t + 1 < NCHUNK)
            def _():
                wait_idx(ns)
                fire_gather(nb, ns)

            wait_midx(s)

            @plsc.parallel_loop(0, IB // L)
            def _(g):
                mcs[b][pl.ds(g * L, L)] = mcnt_r[s, pl.ds(g * L, L)]
            fire_mscatter(b, s)

            wait_gather(b)
            scale(b, s)
            fire_scatter(b, s)
        return 0
    lax.fori_loop(0, NCHUNK // NIDX, body, 0)

    for t in range(NCHUNK - NBUF, NCHUNK):
        wait_scatter(t % NBUF)
        wait_mscatter(t % NBUF)

    plsc.subcore_barrier()

    dbase = sid * ROWS_PER_TILE
    pltpu.sync_copy(acc_c.at[pl.ds(dbase, ROWS_PER_TILE)], tbuf)

    def dump_pair(j, _):
        for b in range(NBUF):
            k = j * NBUF + b
            pltpu.sync_copy(acc_e.at[pl.ds(dbase + k * IB, IB)], gb[b])

            @plsc.parallel_loop(0, IB // L, unroll=2)
            def _(g):
                tv = tbuf[pl.ds(k * IB + g * L, L)]
                rv = ones / tv
                for e in range(L):
                    c = rv[e] * ones
                    r = g * L + e
                    for q in range(QF):
                        db[g * 2 + e // 8, e % 8, pl.ds(q * L, L)] = (
                            gb[b][r, pl.ds(q * L, L)] * c)
            pltpu.sync_copy(
                db, out_e.at[cid].at[pl.ds(sid * (ROWS_PER_TILE // 8)
                                           + k * (IB // 8), IB // 8)])
        return 0
    lax.fori_loop(0, NPIECE // NBUF, dump_pair, 0)


def _combine_body(pe_ref, o_ref):
    s = pe_ref[0] + pe_ref[1]
    o_ref[...] = s.reshape(s.shape[0] * 8, 128)[:, :NUM_FEATURES]


@jax.jit
def kernel(matrix, counts, row_ids, col_ids):
    colh = col_ids.astype(jnp.int32).reshape(ROWS_TOTAL, IB)
    rowh = row_ids.astype(jnp.int32).reshape(ROWS_TOTAL, IB)
    cnth = counts.reshape(ROWS_TOTAL, IB)

    mesh = plsc.VectorSubcoreMesh(core_axis_name="c", subcore_axis_name="s")
    sc = pl.kernel(
        _sc_body,
        out_type=jax.ShapeDtypeStruct((NC, BATCH // 8, 8, 128),
                                      jnp.float32),
        mesh=mesh,
        compiler_params=pltpu.CompilerParams(
            use_tc_tiling_on_sc=False, needs_layout_passes=False),
        scratch_types=(
            [
                pltpu.VMEM_SHARED((BATCH, NUM_FEATURES), jnp.float32),
                pltpu.VMEM_SHARED((BATCH,), jnp.float32),
                pltpu.VMEM((NIDX, IB), jnp.int32),
                pltpu.VMEM((NIDX, IB), jnp.int32),
                pltpu.VMEM((NIDX, IB), jnp.float32),
                pltpu.VMEM((NIDX, IB), jnp.int32),
                pltpu.VMEM((NIDX, IB), jnp.float32),
                pltpu.VMEM((ROWS_PER_TILE,), jnp.float32),
                pltpu.VMEM((IB // 8, 8, 128), jnp.float32),
            ]
            + [pltpu.VMEM((IB, NUM_FEATURES), jnp.float32)] * NBUF
            + [pltpu.VMEM((IB, NUM_FEATURES), jnp.float32)] * NBUF
            + [pltpu.VMEM((IB,), jnp.float32)] * NBUF
            + [pltpu.VMEM((IB,), jnp.float32)] * NBUF
            + [pltpu.SemaphoreType.DMA] * (3 * NBUF + 2 * NIDX)
        ),
    )
    part_e = sc(matrix, colh, rowh, cnth)

    rows_blk = 2048
    out = pl.pallas_call(
        _combine_body,
        grid=(BATCH // rows_blk,),
        in_specs=[
            pl.BlockSpec((NC, rows_blk // 8, 8, 128),
                         lambda i: (0, i, 0, 0)),
        ],
        out_specs=pl.BlockSpec((rows_blk, NUM_FEATURES), lambda i: (i, 0)),
        out_shape=jax.ShapeDtypeStruct((BATCH, NUM_FEATURES), jnp.float32),
    )(part_e)
    return out
